# baseline (speedup 1.0000x reference)
import functools

import jax
import jax.numpy as jnp
from jax import lax
from jax.experimental import pallas as pl
from jax.experimental.pallas import tpu as pltpu
from jax.experimental.pallas import tpu_sc as plsc

_VOCAB = 100000
_MAXLEN = 2048
_D = 64
_B = 4
_S = 2048
_BT = _B * _S

_info = plsc.get_sparse_core_info()
_NC = _info.num_cores
_NS = _info.num_subcores
_L = _info.num_lanes
_NW = _NC * _NS
_BPW = _BT // _NW
_CHUNK = 128
_NCH = _BPW // _CHUNK

_mesh = plsc.VectorSubcoreMesh(core_axis_name="c", subcore_axis_name="s")


@functools.partial(
    pl.kernel,
    out_type=jax.ShapeDtypeStruct((_BT, _D), jnp.float32),
    mesh=_mesh,
    compiler_params=pltpu.CompilerParams(use_tc_tiling_on_sc=False),
    scratch_types=[
        pltpu.VMEM((_NCH, _CHUNK), jnp.int32),
        pltpu.VMEM((_BPW, _D), jnp.float32),
        pltpu.VMEM((_BPW, _D), jnp.float32),
        pltpu.SemaphoreType.DMA,
    ],
)
def _embed(x_hbm, tok_hbm, pos_hbm, out_hbm, idx_v, tok_v, pos_v, sem):
    wid = lax.axis_index("s") * _NC + lax.axis_index("c")
    base = wid * _BPW
    pos_base = lax.rem(base, _S)

    pltpu.sync_copy(x_hbm.at[pl.ds(wid * _NCH, _NCH)], idx_v)

    copies = [
        pltpu.async_copy(
            tok_hbm.at[idx_v.at[j]],
            tok_v.at[pl.ds(j * _CHUNK, _CHUNK)],
            sem,
        )
        for j in range(_NCH)
    ]
    pltpu.sync_copy(pos_hbm.at[pl.ds(pos_base, _BPW)], pos_v)
    for cp in copies:
        cp.wait()

    def row_body(r, _):
        for j in range(_D // _L):
            plsc.addupdate(
                tok_v.at[r, pl.ds(j * _L, _L)],
                pos_v[r, pl.ds(j * _L, _L)],
            )
        return ()

    lax.fori_loop(0, _BPW, row_body, (), unroll=4)

    pltpu.sync_copy(tok_v, out_hbm.at[pl.ds(base, _BPW)])


def kernel(x, token_table, pos_table):
    xf = x.reshape(_NW * _NCH, _CHUNK).astype(jnp.int32)
    out = _embed(xf, token_table, pos_table)
    return out.reshape(_B, _S, _D)


# speedup vs baseline: 2.1084x; 2.1084x over previous
import functools

import jax
import jax.numpy as jnp
from jax import lax
from jax.experimental import pallas as pl
from jax.experimental.pallas import tpu as pltpu
from jax.experimental.pallas import tpu_sc as plsc

_VOCAB = 100000
_D = 64
_B = 4
_S = 2048

_info = plsc.get_sparse_core_info()
_NC = _info.num_cores
_NS = _info.num_subcores
_L = _info.num_lanes
_NW = _NC * _NS
_DPW = _D // _NW

_mesh = plsc.VectorSubcoreMesh(core_axis_name="c", subcore_axis_name="s")


@functools.partial(
    pl.kernel,
    out_type=jax.ShapeDtypeStruct((_B, _D, _S), jnp.float32),
    mesh=_mesh,
    compiler_params=pltpu.CompilerParams(needs_layout_passes=False),
    scratch_types=[
        pltpu.VMEM((_VOCAB,), jnp.float32),
        pltpu.VMEM((_B, _S), jnp.int32),
        pltpu.VMEM((_S,), jnp.float32),
        pltpu.VMEM((_B, _S), jnp.float32),
        pltpu.SemaphoreType.DMA,
    ],
)
def _embed(x_hbm, tokT_hbm, posT_hbm, out_hbm, row_v, idx_v, pos_v, out_v, sem):
    wid = lax.axis_index("s") * _NC + lax.axis_index("c")

    pltpu.sync_copy(x_hbm, idx_v)

    def do_dim(k, _):
        d = wid * _DPW + k
        pltpu.sync_copy(tokT_hbm.at[d], row_v)
        pltpu.sync_copy(posT_hbm.at[d], pos_v)

        def svec(i, _):
            base = i * _L
            pv = pos_v[pl.ds(base, _L)]
            for b in range(_B):
                ids = idx_v[b, pl.ds(base, _L)]
                g = plsc.load_gather(row_v, [ids])
                out_v[b, pl.ds(base, _L)] = g + pv
            return ()

        lax.fori_loop(0, _S // _L, svec, (), unroll=4)
        for b in range(_B):
            pltpu.sync_copy(out_v.at[b], out_hbm.at[b, d])
        return ()

    lax.fori_loop(0, _DPW, do_dim, (), unroll=1)


def kernel(x, token_table, pos_table):
    out = _embed(x.astype(jnp.int32), token_table.T, pos_table.T)
    return out.transpose(0, 2, 1)


# speedup vs baseline: 2.1194x; 1.0052x over previous
import functools

import jax
import jax.numpy as jnp
from jax import lax
from jax.experimental import pallas as pl
from jax.experimental.pallas import tpu as pltpu
from jax.experimental.pallas import tpu_sc as plsc

_VOCAB = 100000
_D = 64
_B = 4
_S = 2048

_info = plsc.get_sparse_core_info()
_NC = _info.num_cores
_NS = _info.num_subcores
_L = _info.num_lanes
_NW = _NC * _NS
_DPW = _D // _NW

_mesh = plsc.VectorSubcoreMesh(core_axis_name="c", subcore_axis_name="s")


@functools.partial(
    pl.kernel,
    out_type=jax.ShapeDtypeStruct((_B, _D, _S), jnp.float32),
    mesh=_mesh,
    compiler_params=pltpu.CompilerParams(needs_layout_passes=False),
    scratch_types=[
        pltpu.VMEM((_VOCAB,), jnp.float32),
        pltpu.VMEM((_B, _S), jnp.int32),
        pltpu.VMEM((_DPW, _S), jnp.float32),
        pltpu.VMEM((_DPW, _B, _S), jnp.float32),
        pltpu.SemaphoreType.DMA,
    ],
)
def _embed(x_hbm, tokT_hbm, posT_hbm, out_hbm, row_v, idx_v, pos_v, out_v, sem):
    wid = lax.axis_index("s") * _NC + lax.axis_index("c")
    d0 = wid * _DPW

    cp_row0 = pltpu.async_copy(tokT_hbm.at[d0], row_v, sem)
    cp_idx = pltpu.async_copy(x_hbm, idx_v, sem)
    cp_pos = pltpu.async_copy(posT_hbm.at[pl.ds(d0, _DPW)], pos_v, sem)

    def compute(k):
        def svec(i, _):
            base = i * _L
            pv = pos_v[k, pl.ds(base, _L)]
            for b in range(_B):
                ids = idx_v[b, pl.ds(base, _L)]
                g = plsc.load_gather(row_v, [ids])
                out_v[k, b, pl.ds(base, _L)] = g + pv
            return ()

        lax.fori_loop(0, _S // _L, svec, (), unroll=8)

    cp_idx.wait()
    cp_pos.wait()
    cp_row0.wait()
    compute(0)
    outs0 = [
        pltpu.async_copy(out_v.at[0, b], out_hbm.at[b, d0], sem)
        for b in range(_B)
    ]
    cp_row1 = pltpu.async_copy(tokT_hbm.at[d0 + 1], row_v, sem)
    cp_row1.wait()
    compute(1)
    for b in range(_B):
        pltpu.sync_copy(out_v.at[1, b], out_hbm.at[b, d0 + 1])
    for cp in outs0:
        cp.wait()


def kernel(x, token_table, pos_table):
    out = _embed(x.astype(jnp.int32), token_table.T, pos_table.T)
    return out.transpose(0, 2, 1)
